# baseline (speedup 1.0000x reference)
import functools

import jax
import jax.numpy as jnp
from jax import lax
from jax.experimental import pallas as pl
from jax.experimental.pallas import tpu as pltpu
from jax.experimental.pallas import tpu_sc as plsc


def _build_gather(B, V, D):
  info = plsc.get_sparse_core_info()
  num_workers = info.num_cores * info.num_subcores
  assert B % num_workers == 0
  b_per_w = B // num_workers

  mesh = plsc.VectorSubcoreMesh(core_axis_name="c", subcore_axis_name="s")

  @functools.partial(
      pl.kernel,
      mesh=mesh,
      out_type=jax.ShapeDtypeStruct((B, D), jnp.float32),
      scratch_types=[
          pltpu.VMEM((b_per_w,), jnp.int32),
          pltpu.VMEM((b_per_w, D), jnp.float32),
          pltpu.SemaphoreType.DMA,
      ],
  )
  def gather_kernel(idx_hbm, table_hbm, out_hbm, idx_v, rows_v, sem):
    wid = lax.axis_index("s") * info.num_cores + lax.axis_index("c")
    base = wid * b_per_w
    pltpu.sync_copy(idx_hbm.at[pl.ds(base, b_per_w)], idx_v)
    copies = []
    for g in range(b_per_w // 16):
      vec = idx_v[pl.ds(g * 16, 16)]
      for j in range(16):
        i = g * 16 + j
        r = vec[j]
        copies.append(
            pltpu.make_async_copy(table_hbm.at[r], rows_v.at[i], sem))
    for c in copies:
      c.start()
    for c in copies:
      c.wait()
    pltpu.sync_copy(rows_v, out_hbm.at[pl.ds(base, b_per_w)])

  return gather_kernel


def kernel(inputs, table):
  B = inputs.shape[0]
  V, D = table.shape
  idx = inputs.astype(jnp.int32)
  return _build_gather(B, V, D)(idx, table)
